# baseline (speedup 1.0000x reference)
import jax
import jax.numpy as jnp
from jax.experimental import pallas as pl
from jax.experimental.pallas import tpu as pltpu

_BM = 256
_NBUF = 4


def _mm_manual(a_hbm, v_ref, o_ref, bufs, sems):
    steps = a_hbm.shape[0] // _BM

    def start_copy(slot, blk):
        pltpu.make_async_copy(
            a_hbm.at[pl.ds(blk * _BM, _BM), :], bufs.at[slot], sems.at[slot]
        ).start()

    for s in range(_NBUF):
        start_copy(s, s)

    v = v_ref[...]

    def step(i, carry):
        slot = jax.lax.rem(i, _NBUF)
        pltpu.make_async_copy(
            a_hbm.at[pl.ds(i * _BM, _BM), :], bufs.at[slot], sems.at[slot]
        ).wait()
        o_ref[pl.ds(i * _BM, _BM), :] = jnp.dot(
            bufs[slot], v, preferred_element_type=jnp.float32
        )

        @pl.when(i + _NBUF < steps)
        def _next():
            start_copy(slot, i + _NBUF)

        return carry

    jax.lax.fori_loop(0, steps, step, 0)


def kernel(matrix, matrix_batch):
    m, k = matrix.shape
    b, _, f = matrix_batch.shape
    n = b * f
    vectors = jnp.swapaxes(matrix_batch, 0, 1).reshape(k, n)

    out = pl.pallas_call(
        _mm_manual,
        in_specs=[
            pl.BlockSpec(memory_space=pltpu.MemorySpace.HBM),
            pl.BlockSpec(memory_space=pltpu.MemorySpace.VMEM),
        ],
        out_specs=pl.BlockSpec(memory_space=pltpu.MemorySpace.VMEM),
        out_shape=jax.ShapeDtypeStruct((m, n), jnp.float32),
        scratch_shapes=[
            pltpu.VMEM((_NBUF, _BM, k), jnp.float32),
            pltpu.SemaphoreType.DMA((_NBUF,)),
        ],
    )(matrix, vectors)

    return jnp.swapaxes(out.reshape(m, b, f), 0, 1)
